# baseline (speedup 1.0000x reference)
import functools

import jax
import jax.numpy as jnp
from jax import lax
from jax.experimental import pallas as pl
from jax.experimental.pallas import tpu as pltpu
from jax.experimental.pallas import tpu_sc as plsc

_ROWS = 128
_N = 32768
_K = 256
_L = 16
_NV = _N // _L
_NWORK = 32
_RPW = _ROWS // _NWORK
_NBINS = 256


def _f32_key(v):
  bits = plsc.bitcast(v, jnp.int32)
  key = bits ^ ((bits >> 31) | jnp.int32(-2147483648))
  return plsc.bitcast(key, jnp.uint32)


def _lperm(x, perm):
  return jnp.take_along_axis(x, perm, axis=0)


def _sc_body(x_hbm, o_hbm, row_v, cand_v, hist_v, outi_v):
  cid = lax.axis_index("c")
  sid = lax.axis_index("s")
  wid = sid * 2 + cid
  lane = lax.iota(jnp.int32, _L)
  ones = jnp.ones((_L,), jnp.int32)
  zeros = jnp.zeros((_L,), jnp.int32)
  laneoff = lane * _NBINS

  def _popcnt_v(mask):
    return plsc.all_reduce_population_count(mask)

  def _zero_hist(b, _):
    hist_v[pl.ds(b * _L, _L)] = zeros
    return 0

  lax.fori_loop(0, _NBINS, _zero_hist, 0)

  def find_bin(need):
    need_v = zeros + need
    lane15 = jnp.full((_L,), 15, jnp.int32)

    def fb(g, carry):
      cum, b0, nbelow = carry
      base = g * _L
      acc = zeros
      for l in range(_L):
        sl = pl.ds(l * _NBINS + base, _L)
        acc = acc + hist_v[sl]
        hist_v[sl] = zeros
      csum = plsc.cumsum(acc)
      tot = jnp.take_along_axis(csum, lane15, axis=0)
      crossed = (cum + csum) >= need_v
      ffs = plsc.all_reduce_ffs(crossed)
      nb = cum + jnp.take_along_axis(csum - acc, ffs, axis=0)
      hit = (b0 < 0) & ((cum + tot) >= need_v)
      b0 = jnp.where(hit, base + ffs, b0)
      nbelow = jnp.where(hit, nb, nbelow)
      return (cum + tot, b0, nbelow)

    minus1 = jnp.full((_L,), -1, jnp.int32)
    _, b0_v, nbelow_v = lax.fori_loop(0, _L, fb, (zeros, minus1, zeros))
    return b0_v[0], nbelow_v[0]

  def do_row(r, _):
    row = wid * _RPW + r
    pltpu.sync_copy(x_hbm.at[row], row_v)

    def pa(i, c):
      key = _f32_key(row_v[pl.ds(i * _L, _L)])
      dig = (key >> 24).astype(jnp.int32)
      plsc.addupdate_scatter(hist_v, [laneoff + dig], ones)
      return c

    lax.fori_loop(0, _NV, pa, 0)
    b0, _ = find_bin(jnp.int32(_K))

    def pb(i, acc):
      key = _f32_key(row_v[pl.ds(i * _L, _L)])
      keep = (key >> 24).astype(jnp.int32) <= b0
      idxv = i * _L + lane
      pos = acc + plsc.cumsum(keep.astype(jnp.int32)) - 1
      plsc.store_scatter(cand_v, [pos], idxv, mask=keep)
      return acc + _popcnt_v(keep)

    mcount = lax.fori_loop(0, _NV, pb, zeros)[0]

    path = b0.astype(jnp.uint32)
    n_lt_full = jnp.int32(0)
    for shift in (16, 8, 0):
      nvec = (mcount + _L - 1) // _L

      def lv1(i, nlt, shift=shift, path=path, mcount=mcount):
        valid = (i * _L + lane) < mcount
        idxv = cand_v[pl.ds(i * _L, _L)]
        key = _f32_key(plsc.load_gather(row_v, [idxv], mask=valid))
        pre = key >> (shift + 8)
        is_lt = valid & (pre < path)
        is_eq = valid & (pre == path)
        dig = ((key >> shift) & 0xFF).astype(jnp.int32)
        plsc.addupdate_scatter(hist_v, [laneoff + dig], ones, mask=is_eq)
        return nlt + _popcnt_v(is_lt)

      n_lt = lax.fori_loop(0, nvec, lv1, zeros)[0]
      bl, nbelow = find_bin(_K - n_lt)
      path = (path << 8) | bl.astype(jnp.uint32)
      n_lt_full = n_lt + nbelow

      def lv2(i, acc, shift=shift, path=path, mcount=mcount):
        valid = (i * _L + lane) < mcount
        idxv = cand_v[pl.ds(i * _L, _L)]
        key = _f32_key(plsc.load_gather(row_v, [idxv], mask=valid))
        keep = valid & ((key >> shift) <= path)
        pos = acc + plsc.cumsum(keep.astype(jnp.int32)) - 1
        plsc.store_scatter(cand_v, [pos], idxv, mask=keep)
        return acc + _popcnt_v(keep)

      mcount = lax.fori_loop(0, nvec, lv2, zeros)[0]

    tkey = path
    need_ties = _K - n_lt_full
    nvec = (mcount + _L - 1) // _L

    def trim(i, carry):
      acc, teq = carry
      valid = (i * _L + lane) < mcount
      idxv = cand_v[pl.ds(i * _L, _L)]
      key = _f32_key(plsc.load_gather(row_v, [idxv], mask=valid))
      is_lt = valid & (key < tkey)
      is_eq = valid & (key == tkey)
      erank = plsc.cumsum(is_eq.astype(jnp.int32))
      keep = is_lt | (is_eq & ((teq + erank) <= need_ties))
      pos = acc + plsc.cumsum(keep.astype(jnp.int32)) - 1
      plsc.store_scatter(cand_v, [pos], idxv, mask=keep)
      return (acc + _popcnt_v(keep), teq + _popcnt_v(is_eq))

    lax.fori_loop(0, nvec, trim, (zeros, zeros))

    kv = []
    iv = []
    for i in range(16):
      idxv = cand_v[pl.ds(i * _L, _L)]
      kv.append(_f32_key(plsc.load_gather(row_v, [idxv])))
      iv.append(idxv)

    def ce_vreg(i, j, bsz):
      ka, ia, kb, ib = kv[i], iv[i], kv[j], iv[j]
      if bsz >= 16:
        dirm = ((lane * 16 + i) & bsz) == 0
      else:
        dirm = ((i & bsz) == 0)
        dirm = jnp.full((_L,), dirm, jnp.bool_)
      lt = (ka < kb) | ((ka == kb) & (ia < ib))
      keep_a = lt == dirm
      kv[i] = jnp.where(keep_a, ka, kb)
      iv[i] = jnp.where(keep_a, ia, ib)
      kv[j] = jnp.where(keep_a, kb, ka)
      iv[j] = jnp.where(keep_a, ib, ia)

    def ce_lane(i, dd, bsz):
      perm = lane ^ dd
      k0, i0 = kv[i], iv[i]
      kp = _lperm(k0, perm)
      ip = _lperm(i0, perm)
      lt = (k0 < kp) | ((k0 == kp) & (i0 < ip))
      dirm = ((lane * 16 + i) & bsz) == 0
      wantmin = ((lane & dd) == 0) == dirm
      keep = lt == wantmin
      kv[i] = jnp.where(keep, k0, kp)
      iv[i] = jnp.where(keep, i0, ip)

    bsz = 2
    while bsz <= 256:
      d = bsz // 2
      while d >= 1:
        if d < 16:
          for i in range(16):
            if (i & d) == 0:
              ce_vreg(i, i | d, bsz)
        else:
          dd = d // 16
          for i in range(16):
            ce_lane(i, dd, bsz)
        d //= 2
      bsz *= 2

    for i in range(16):
      plsc.store_scatter(outi_v, [lane * 16 + i], iv[i])
    pltpu.sync_copy(outi_v, o_hbm.at[row])
    return 0

  lax.fori_loop(0, _RPW, do_row, 0)


@jax.jit
def _topk_small_idx(x):
  mesh = plsc.VectorSubcoreMesh(core_axis_name="c", subcore_axis_name="s")
  return pl.kernel(
      _sc_body,
      out_type=jax.ShapeDtypeStruct((_ROWS, _K), jnp.int32),
      mesh=mesh,
      compiler_params=pltpu.CompilerParams(needs_layout_passes=False),
      scratch_types=[
          pltpu.VMEM((_N,), jnp.float32),
          pltpu.VMEM((_N + _L,), jnp.int32),
          pltpu.VMEM((_NBINS * _L,), jnp.int32),
          pltpu.VMEM((_K,), jnp.int32),
      ],
  )(x)


def kernel(x, k):
  del k
  return _topk_small_idx(x)


# speedup vs baseline: 1.1596x; 1.1596x over previous
import functools

import jax
import jax.numpy as jnp
from jax import lax
from jax.experimental import pallas as pl
from jax.experimental.pallas import tpu as pltpu
from jax.experimental.pallas import tpu_sc as plsc

_ROWS = 128
_N = 32768
_K = 256
_L = 16
_NV = _N // _L
_NWORK = 32
_RPW = _ROWS // _NWORK
_NBINS = 256


def _f32_key(v):
  bits = plsc.bitcast(v, jnp.int32)
  key = bits ^ ((bits >> 31) | jnp.int32(-2147483648))
  return plsc.bitcast(key, jnp.uint32)


def _lperm(x, perm):
  return jnp.take_along_axis(x, perm, axis=0)


def _sc_body(x_hbm, o_hbm, row_v, cand_v, hist_v, outi_v):
  cid = lax.axis_index("c")
  sid = lax.axis_index("s")
  wid = sid * 2 + cid
  lane = lax.iota(jnp.int32, _L)
  ones = jnp.ones((_L,), jnp.int32)
  zeros = jnp.zeros((_L,), jnp.int32)
  laneoff = lane * _NBINS

  def _popcnt(mask):
    return plsc.all_reduce_population_count(mask)[0]

  def _zero_hist(b, _):
    hist_v[pl.ds(b * _L, _L)] = zeros
    return 0

  lax.fori_loop(0, _NBINS, _zero_hist, 0)

  def find_bin(need):
    def fb(g, carry):
      cum, b0, nbelow = carry
      base = g * _L
      acc = zeros
      for l in range(_L):
        sl = pl.ds(l * _NBINS + base, _L)
        acc = acc + hist_v[sl]
        hist_v[sl] = zeros
      csum = plsc.cumsum(acc)
      tot = csum[15]
      crossed = (cum + csum) >= need
      ffs = plsc.all_reduce_ffs(crossed)
      nb = cum + jnp.take_along_axis(csum - acc, ffs, axis=0)[0]
      hit = (b0 < 0) & ((cum + tot) >= need)
      b0 = jnp.where(hit, base + ffs[0], b0)
      nbelow = jnp.where(hit, nb, nbelow)
      return (cum + tot, b0, nbelow)

    _, b0, nbelow = lax.fori_loop(
        0, _L, fb, (jnp.int32(0), jnp.int32(-1), jnp.int32(0)))
    return b0, nbelow

  def do_row(r, _):
    row = wid * _RPW + r
    with jax.named_scope("stage_dma_in"):
      pltpu.sync_copy(x_hbm.at[row], row_v)

    def pa(i, c):
      key = _f32_key(row_v[pl.ds(i * _L, _L)])
      dig = (key >> 24).astype(jnp.int32)
      plsc.addupdate_scatter(hist_v, [laneoff + dig], ones)
      return c

    with jax.named_scope("stage_pa"):
      lax.fori_loop(0, _NV, pa, 0)
    with jax.named_scope("stage_findbin"):
      b0, _ = find_bin(jnp.int32(_K))

    def pb(i, w):
      key = _f32_key(row_v[pl.ds(i * _L, _L)])
      keep = (key >> 24).astype(jnp.int32) <= b0
      idxv = i * _L + lane
      plsc.store_compressed(cand_v.at[pl.ds(w, _L)], idxv, mask=keep)
      return w + _popcnt(keep)

    with jax.named_scope("stage_pb"):
      mcount = lax.fori_loop(0, _NV, pb, jnp.int32(0))

    path = b0.astype(jnp.uint32)
    n_lt_full = jnp.int32(0)
    for shift in (16, 8, 0):
      nvec = (mcount + _L - 1) // _L

      def lv1(i, nlt, shift=shift, path=path, mcount=mcount):
        valid = (i * _L + lane) < mcount
        idxv = cand_v[pl.ds(i * _L, _L)]
        key = _f32_key(plsc.load_gather(row_v, [idxv], mask=valid))
        pre = key >> (shift + 8)
        is_lt = valid & (pre < path)
        is_eq = valid & (pre == path)
        dig = ((key >> shift) & 0xFF).astype(jnp.int32)
        plsc.addupdate_scatter(hist_v, [laneoff + dig], ones, mask=is_eq)
        return nlt + _popcnt(is_lt)

      with jax.named_scope("stage_lv1"):
        n_lt = lax.fori_loop(0, nvec, lv1, jnp.int32(0))
      bl, nbelow = find_bin(_K - n_lt)
      path = (path << 8) | bl.astype(jnp.uint32)
      n_lt_full = n_lt + nbelow

      def lv2(i, w, shift=shift, path=path, mcount=mcount):
        valid = (i * _L + lane) < mcount
        idxv = cand_v[pl.ds(i * _L, _L)]
        key = _f32_key(plsc.load_gather(row_v, [idxv], mask=valid))
        keep = valid & ((key >> shift) <= path)
        plsc.store_compressed(cand_v.at[pl.ds(w, _L)], idxv, mask=keep)
        return w + _popcnt(keep)

      with jax.named_scope("stage_lv2"):
        mcount = lax.fori_loop(0, nvec, lv2, jnp.int32(0))

    tkey = path
    need_ties = _K - n_lt_full
    nvec = (mcount + _L - 1) // _L

    def trim(i, carry):
      w, teq = carry
      valid = (i * _L + lane) < mcount
      idxv = cand_v[pl.ds(i * _L, _L)]
      key = _f32_key(plsc.load_gather(row_v, [idxv], mask=valid))
      is_lt = valid & (key < tkey)
      is_eq = valid & (key == tkey)
      erank = plsc.cumsum(is_eq.astype(jnp.int32))
      keep = is_lt | (is_eq & ((teq + erank) <= need_ties))
      plsc.store_compressed(cand_v.at[pl.ds(w, _L)], idxv, mask=keep)
      return (w + _popcnt(keep), teq + _popcnt(is_eq))

    with jax.named_scope("stage_trim"):
      lax.fori_loop(0, nvec, trim, (jnp.int32(0), jnp.int32(0)))

    kv = []
    iv = []
    for i in range(16):
      idxv = cand_v[pl.ds(i * _L, _L)]
      kv.append(_f32_key(plsc.load_gather(row_v, [idxv])))
      iv.append(idxv)

    def ce_vreg(i, j, bsz):
      ka, ia, kb, ib = kv[i], iv[i], kv[j], iv[j]
      if bsz >= 16:
        dirm = ((lane * 16 + i) & bsz) == 0
      else:
        dirm = ((i & bsz) == 0)
        dirm = jnp.full((_L,), dirm, jnp.bool_)
      lt = (ka < kb) | ((ka == kb) & (ia < ib))
      keep_a = lt == dirm
      kv[i] = jnp.where(keep_a, ka, kb)
      iv[i] = jnp.where(keep_a, ia, ib)
      kv[j] = jnp.where(keep_a, kb, ka)
      iv[j] = jnp.where(keep_a, ib, ia)

    def ce_lane(i, dd, bsz):
      perm = lane ^ dd
      k0, i0 = kv[i], iv[i]
      kp = _lperm(k0, perm)
      ip = _lperm(i0, perm)
      lt = (k0 < kp) | ((k0 == kp) & (i0 < ip))
      dirm = ((lane * 16 + i) & bsz) == 0
      wantmin = ((lane & dd) == 0) == dirm
      keep = lt == wantmin
      kv[i] = jnp.where(keep, k0, kp)
      iv[i] = jnp.where(keep, i0, ip)

    _sort_scope = jax.named_scope("stage_sort"); _sort_scope.__enter__()
    bsz = 2
    while bsz <= 256:
      d = bsz // 2
      while d >= 1:
        if d < 16:
          for i in range(16):
            if (i & d) == 0:
              ce_vreg(i, i | d, bsz)
        else:
          dd = d // 16
          for i in range(16):
            ce_lane(i, dd, bsz)
        d //= 2
      bsz *= 2

    with jax.named_scope("stage_out"):
      for i in range(16):
        plsc.store_scatter(outi_v, [lane * 16 + i], iv[i])
      pltpu.sync_copy(outi_v, o_hbm.at[row])
    return 0

  lax.fori_loop(0, _RPW, do_row, 0)


@jax.jit
def _topk_small_idx(x):
  mesh = plsc.VectorSubcoreMesh(core_axis_name="c", subcore_axis_name="s")
  return pl.kernel(
      _sc_body,
      out_type=jax.ShapeDtypeStruct((_ROWS, _K), jnp.int32),
      mesh=mesh,
      compiler_params=pltpu.CompilerParams(needs_layout_passes=False),
      scratch_types=[
          pltpu.VMEM((_N,), jnp.float32),
          pltpu.VMEM((_N + _L,), jnp.int32),
          pltpu.VMEM((_NBINS * _L,), jnp.int32),
          pltpu.VMEM((_K,), jnp.int32),
      ],
  )(x)


def kernel(x, k):
  del k
  return _topk_small_idx(x)


# speedup vs baseline: 1.2205x; 1.0525x over previous
import functools

import jax
import jax.numpy as jnp
from jax import lax
from jax.experimental import pallas as pl
from jax.experimental.pallas import tpu as pltpu
from jax.experimental.pallas import tpu_sc as plsc

_ROWS = 128
_N = 32768
_K = 256
_L = 16
_NV = _N // _L
_NWORK = 32
_RPW = _ROWS // _NWORK
_NBINS = 256


def _f32_key(v):
  bits = plsc.bitcast(v, jnp.int32)
  key = bits ^ ((bits >> 31) | jnp.int32(-2147483648))
  return plsc.bitcast(key, jnp.uint32)


def _lperm(x, perm):
  return jnp.take_along_axis(x, perm, axis=0)


def _sc_body(x_hbm, o_hbm, row_v, cand_v, hist_v, outi_v):
  cid = lax.axis_index("c")
  sid = lax.axis_index("s")
  wid = sid * 2 + cid
  lane = lax.iota(jnp.int32, _L)
  ones = jnp.ones((_L,), jnp.int32)
  zeros = jnp.zeros((_L,), jnp.int32)
  laneoff = lane * _NBINS

  def _popcnt(mask):
    return plsc.all_reduce_population_count(mask)[0]

  def _zero_hist(b, _):
    hist_v[pl.ds(b * _L, _L)] = zeros
    return 0

  lax.fori_loop(0, _NBINS, _zero_hist, 0)

  lane15 = jnp.full((_L,), 15, jnp.int32)

  def find_bin(need):
    need_v = zeros + need

    def p1(g, gtot):
      acc = zeros
      for t in range(_L):
        acc = acc + hist_v[pl.ds((g * _L + t) * _L, _L)]
      tsp = jnp.take_along_axis(plsc.cumsum(acc), lane15, axis=0)
      return jnp.where(lane == g, tsp, gtot)

    gtot = lax.fori_loop(0, _L, p1, zeros)
    gcs = plsc.cumsum(gtot)
    gxv = plsc.all_reduce_ffs(gcs >= need_v)
    gx = gxv[0]
    cum0_v = jnp.take_along_axis(gcs - gtot, gxv, axis=0)

    def p2(g, btot):
      def resolve():
        upd = btot
        for t in range(_L):
          v = hist_v[pl.ds((g * _L + t) * _L, _L)]
          tsp = jnp.take_along_axis(plsc.cumsum(v), lane15, axis=0)
          upd = jnp.where(lane == t, tsp, upd)
        return upd

      upd = lax.cond(g == gx, resolve, lambda: btot)
      for t in range(_L):
        hist_v[pl.ds((g * _L + t) * _L, _L)] = zeros
      return upd

    btot = lax.fori_loop(0, _L, p2, zeros)
    bcs = plsc.cumsum(btot)
    bfv = plsc.all_reduce_ffs((cum0_v + bcs) >= need_v)
    b0 = gx * _L + bfv[0]
    nbelow_v = cum0_v + jnp.take_along_axis(bcs - btot, bfv, axis=0)
    return b0, nbelow_v[0]

  def do_row(r, _):
    row = wid * _RPW + r
    with jax.named_scope("stage_dma_in"):
      pltpu.sync_copy(x_hbm.at[row], row_v)

    def pa(i, c):
      key = _f32_key(row_v[pl.ds(i * _L, _L)])
      dig = (key >> 24).astype(jnp.int32)
      plsc.addupdate_scatter(hist_v, [dig * _L + lane], ones)
      return c

    with jax.named_scope("stage_pa"):
      lax.fori_loop(0, _NV, pa, 0)
    with jax.named_scope("stage_findbin"):
      b0, _ = find_bin(jnp.int32(_K))

    def pb(i, w):
      key = _f32_key(row_v[pl.ds(i * _L, _L)])
      keep = (key >> 24).astype(jnp.int32) <= b0
      idxv = i * _L + lane
      plsc.store_compressed(cand_v.at[pl.ds(w, _L)], idxv, mask=keep)
      return w + _popcnt(keep)

    with jax.named_scope("stage_pb"):
      mcount = lax.fori_loop(0, _NV, pb, jnp.int32(0))

    path = b0.astype(jnp.uint32)
    n_lt_full = jnp.int32(0)
    for shift in (16, 8, 0):
      nvec = (mcount + _L - 1) // _L

      def lv1(i, nlt, shift=shift, path=path, mcount=mcount):
        valid = (i * _L + lane) < mcount
        idxv = cand_v[pl.ds(i * _L, _L)]
        key = _f32_key(plsc.load_gather(row_v, [idxv], mask=valid))
        pre = key >> (shift + 8)
        is_lt = valid & (pre < path)
        is_eq = valid & (pre == path)
        dig = ((key >> shift) & 0xFF).astype(jnp.int32)
        plsc.addupdate_scatter(hist_v, [dig * _L + lane], ones, mask=is_eq)
        return nlt + _popcnt(is_lt)

      with jax.named_scope("stage_lv1"):
        n_lt = lax.fori_loop(0, nvec, lv1, jnp.int32(0))
      bl, nbelow = find_bin(_K - n_lt)
      path = (path << 8) | bl.astype(jnp.uint32)
      n_lt_full = n_lt + nbelow

      def lv2(i, w, shift=shift, path=path, mcount=mcount):
        valid = (i * _L + lane) < mcount
        idxv = cand_v[pl.ds(i * _L, _L)]
        key = _f32_key(plsc.load_gather(row_v, [idxv], mask=valid))
        keep = valid & ((key >> shift) <= path)
        plsc.store_compressed(cand_v.at[pl.ds(w, _L)], idxv, mask=keep)
        return w + _popcnt(keep)

      with jax.named_scope("stage_lv2"):
        mcount = lax.fori_loop(0, nvec, lv2, jnp.int32(0))

    tkey = path
    need_ties = _K - n_lt_full
    nvec = (mcount + _L - 1) // _L

    def trim(i, carry):
      w, teq = carry
      valid = (i * _L + lane) < mcount
      idxv = cand_v[pl.ds(i * _L, _L)]
      key = _f32_key(plsc.load_gather(row_v, [idxv], mask=valid))
      is_lt = valid & (key < tkey)
      is_eq = valid & (key == tkey)
      erank = plsc.cumsum(is_eq.astype(jnp.int32))
      keep = is_lt | (is_eq & ((teq + erank) <= need_ties))
      plsc.store_compressed(cand_v.at[pl.ds(w, _L)], idxv, mask=keep)
      return (w + _popcnt(keep), teq + _popcnt(is_eq))

    with jax.named_scope("stage_trim"):
      lax.fori_loop(0, nvec, trim, (jnp.int32(0), jnp.int32(0)))

    kv = []
    iv = []
    for i in range(16):
      idxv = cand_v[pl.ds(i * _L, _L)]
      kv.append(_f32_key(plsc.load_gather(row_v, [idxv])))
      iv.append(idxv)

    def ce_vreg(i, j, bsz):
      ka, ia, kb, ib = kv[i], iv[i], kv[j], iv[j]
      if bsz >= 16:
        dirm = ((lane * 16 + i) & bsz) == 0
      else:
        dirm = ((i & bsz) == 0)
        dirm = jnp.full((_L,), dirm, jnp.bool_)
      lt = (ka < kb) | ((ka == kb) & (ia < ib))
      keep_a = lt == dirm
      kv[i] = jnp.where(keep_a, ka, kb)
      iv[i] = jnp.where(keep_a, ia, ib)
      kv[j] = jnp.where(keep_a, kb, ka)
      iv[j] = jnp.where(keep_a, ib, ia)

    def ce_lane(i, dd, bsz):
      perm = lane ^ dd
      k0, i0 = kv[i], iv[i]
      kp = _lperm(k0, perm)
      ip = _lperm(i0, perm)
      lt = (k0 < kp) | ((k0 == kp) & (i0 < ip))
      dirm = ((lane * 16 + i) & bsz) == 0
      wantmin = ((lane & dd) == 0) == dirm
      keep = lt == wantmin
      kv[i] = jnp.where(keep, k0, kp)
      iv[i] = jnp.where(keep, i0, ip)

    _sort_scope = jax.named_scope("stage_sort"); _sort_scope.__enter__()
    bsz = 2
    while bsz <= 256:
      d = bsz // 2
      while d >= 1:
        if d < 16:
          for i in range(16):
            if (i & d) == 0:
              ce_vreg(i, i | d, bsz)
        else:
          dd = d // 16
          for i in range(16):
            ce_lane(i, dd, bsz)
        d //= 2
      bsz *= 2

    with jax.named_scope("stage_out"):
      for i in range(16):
        plsc.store_scatter(outi_v, [lane * 16 + i], iv[i])
      pltpu.sync_copy(outi_v, o_hbm.at[row])
    return 0

  lax.fori_loop(0, _RPW, do_row, 0)


@jax.jit
def _topk_small_idx(x):
  mesh = plsc.VectorSubcoreMesh(core_axis_name="c", subcore_axis_name="s")
  return pl.kernel(
      _sc_body,
      out_type=jax.ShapeDtypeStruct((_ROWS, _K), jnp.int32),
      mesh=mesh,
      compiler_params=pltpu.CompilerParams(needs_layout_passes=False),
      scratch_types=[
          pltpu.VMEM((_N,), jnp.float32),
          pltpu.VMEM((_N + _L,), jnp.int32),
          pltpu.VMEM((_NBINS * _L,), jnp.int32),
          pltpu.VMEM((_K,), jnp.int32),
      ],
  )(x)


def kernel(x, k):
  del k
  return _topk_small_idx(x)


# speedup vs baseline: 3.2545x; 2.6666x over previous
import functools

import jax
import jax.numpy as jnp
from jax import lax
from jax.experimental import pallas as pl
from jax.experimental.pallas import tpu as pltpu
from jax.experimental.pallas import tpu_sc as plsc

_ROWS = 128
_N = 32768
_K = 256
_L = 16
_NV = _N // _L
_NWORK = 32
_RPW = _ROWS // _NWORK
_NBINS = 256


def _f32_key(v):
  bits = plsc.bitcast(v, jnp.int32)
  key = bits ^ ((bits >> 31) | jnp.int32(-2147483648))
  return plsc.bitcast(key, jnp.uint32)


def _lperm(x, perm):
  return jnp.take_along_axis(x, perm, axis=0)


def _sc_body(x_hbm, o_hbm, row_v, cand_v, hist_v, outi_v):
  cid = lax.axis_index("c")
  sid = lax.axis_index("s")
  wid = sid * 2 + cid
  lane = lax.iota(jnp.int32, _L)
  ones = jnp.ones((_L,), jnp.int32)
  zeros = jnp.zeros((_L,), jnp.int32)
  laneoff = lane * _NBINS

  def _popcnt(mask):
    return plsc.all_reduce_population_count(mask)[0]

  def _zero_hist(b, _):
    hist_v[pl.ds(b * _L, _L)] = zeros
    return 0

  lax.fori_loop(0, _NBINS, _zero_hist, 0)

  lane15 = jnp.full((_L,), 15, jnp.int32)

  def find_bin(need):
    need_v = zeros + need

    def p1(g, gtot):
      acc = zeros
      for t in range(_L):
        acc = acc + hist_v[pl.ds((g * _L + t) * _L, _L)]
      tsp = jnp.take_along_axis(plsc.cumsum(acc), lane15, axis=0)
      return jnp.where(lane == g, tsp, gtot)

    gtot = lax.fori_loop(0, _L, p1, zeros)
    gcs = plsc.cumsum(gtot)
    gxv = plsc.all_reduce_ffs(gcs >= need_v)
    gx = gxv[0]
    cum0_v = jnp.take_along_axis(gcs - gtot, gxv, axis=0)

    def p2(g, btot):
      def resolve():
        upd = btot
        for t in range(_L):
          v = hist_v[pl.ds((g * _L + t) * _L, _L)]
          tsp = jnp.take_along_axis(plsc.cumsum(v), lane15, axis=0)
          upd = jnp.where(lane == t, tsp, upd)
        return upd

      upd = lax.cond(g == gx, resolve, lambda: btot)
      for t in range(_L):
        hist_v[pl.ds((g * _L + t) * _L, _L)] = zeros
      return upd

    btot = lax.fori_loop(0, _L, p2, zeros)
    bcs = plsc.cumsum(btot)
    bfv = plsc.all_reduce_ffs((cum0_v + bcs) >= need_v)
    b0 = gx * _L + bfv[0]
    nbelow_v = cum0_v + jnp.take_along_axis(bcs - btot, bfv, axis=0)
    return b0, nbelow_v[0]

  def do_row(r, _):
    row = wid * _RPW + r
    with jax.named_scope("stage_dma_in"):
      pltpu.sync_copy(x_hbm.at[row], row_v)

    with jax.named_scope("stage_pa"):
      @plsc.parallel_loop(0, _NV, unroll=4)
      def pa(i):
        key = _f32_key(row_v[pl.ds(i * _L, _L)])
        dig = (key >> 24).astype(jnp.int32)
        plsc.addupdate_scatter(hist_v, [dig * _L + lane], ones)
    with jax.named_scope("stage_findbin"):
      b0, _ = find_bin(jnp.int32(_K))

    with jax.named_scope("stage_pb"):
      @plsc.parallel_loop(0, _NV, unroll=4, carry=jnp.int32(0))
      def pb(i, w):
        key = _f32_key(row_v[pl.ds(i * _L, _L)])
        keep = (key >> 24).astype(jnp.int32) <= b0
        idxv = i * _L + lane
        plsc.store_compressed(cand_v.at[pl.ds(w, _L)], idxv, mask=keep)
        return w + _popcnt(keep)
      mcount = pb

    path = b0.astype(jnp.uint32)
    n_lt_full = jnp.int32(0)
    for shift in (16, 8, 0):
      nvec = (mcount + _L - 1) // _L

      def lv1(i, nlt, shift=shift, path=path, mcount=mcount):
        valid = (i * _L + lane) < mcount
        idxv = cand_v[pl.ds(i * _L, _L)]
        key = _f32_key(plsc.load_gather(row_v, [idxv], mask=valid))
        pre = key >> (shift + 8)
        is_lt = valid & (pre < path)
        is_eq = valid & (pre == path)
        dig = ((key >> shift) & 0xFF).astype(jnp.int32)
        plsc.addupdate_scatter(hist_v, [dig * _L + lane], ones, mask=is_eq)
        return nlt + _popcnt(is_lt)

      with jax.named_scope("stage_lv1"):
        n_lt = lax.fori_loop(0, nvec, lv1, jnp.int32(0))
      bl, nbelow = find_bin(_K - n_lt)
      path = (path << 8) | bl.astype(jnp.uint32)
      n_lt_full = n_lt + nbelow

      def lv2(i, w, shift=shift, path=path, mcount=mcount):
        valid = (i * _L + lane) < mcount
        idxv = cand_v[pl.ds(i * _L, _L)]
        key = _f32_key(plsc.load_gather(row_v, [idxv], mask=valid))
        keep = valid & ((key >> shift) <= path)
        plsc.store_compressed(cand_v.at[pl.ds(w, _L)], idxv, mask=keep)
        return w + _popcnt(keep)

      with jax.named_scope("stage_lv2"):
        mcount = lax.fori_loop(0, nvec, lv2, jnp.int32(0))

    tkey = path
    need_ties = _K - n_lt_full
    nvec = (mcount + _L - 1) // _L

    def trim(i, carry):
      w, teq = carry
      valid = (i * _L + lane) < mcount
      idxv = cand_v[pl.ds(i * _L, _L)]
      key = _f32_key(plsc.load_gather(row_v, [idxv], mask=valid))
      is_lt = valid & (key < tkey)
      is_eq = valid & (key == tkey)
      erank = plsc.cumsum(is_eq.astype(jnp.int32))
      keep = is_lt | (is_eq & ((teq + erank) <= need_ties))
      plsc.store_compressed(cand_v.at[pl.ds(w, _L)], idxv, mask=keep)
      return (w + _popcnt(keep), teq + _popcnt(is_eq))

    with jax.named_scope("stage_trim"):
      lax.fori_loop(0, nvec, trim, (jnp.int32(0), jnp.int32(0)))

    kv = []
    iv = []
    for i in range(16):
      idxv = cand_v[pl.ds(i * _L, _L)]
      kv.append(_f32_key(plsc.load_gather(row_v, [idxv])))
      iv.append(idxv)

    def ce_vreg(i, j, bsz):
      ka, ia, kb, ib = kv[i], iv[i], kv[j], iv[j]
      if bsz >= 16:
        dirm = ((lane * 16 + i) & bsz) == 0
      else:
        dirm = ((i & bsz) == 0)
        dirm = jnp.full((_L,), dirm, jnp.bool_)
      lt = (ka < kb) | ((ka == kb) & (ia < ib))
      keep_a = lt == dirm
      kv[i] = jnp.where(keep_a, ka, kb)
      iv[i] = jnp.where(keep_a, ia, ib)
      kv[j] = jnp.where(keep_a, kb, ka)
      iv[j] = jnp.where(keep_a, ib, ia)

    def ce_lane(i, dd, bsz):
      perm = lane ^ dd
      k0, i0 = kv[i], iv[i]
      kp = _lperm(k0, perm)
      ip = _lperm(i0, perm)
      lt = (k0 < kp) | ((k0 == kp) & (i0 < ip))
      dirm = ((lane * 16 + i) & bsz) == 0
      wantmin = ((lane & dd) == 0) == dirm
      keep = lt == wantmin
      kv[i] = jnp.where(keep, k0, kp)
      iv[i] = jnp.where(keep, i0, ip)

    _sort_scope = jax.named_scope("stage_sort"); _sort_scope.__enter__()
    bsz = 2
    while bsz <= 256:
      d = bsz // 2
      while d >= 1:
        if d < 16:
          for i in range(16):
            if (i & d) == 0:
              ce_vreg(i, i | d, bsz)
        else:
          dd = d // 16
          for i in range(16):
            ce_lane(i, dd, bsz)
        d //= 2
      bsz *= 2

    with jax.named_scope("stage_out"):
      for i in range(16):
        plsc.store_scatter(outi_v, [lane * 16 + i], iv[i])
      pltpu.sync_copy(outi_v, o_hbm.at[row])
    return 0

  lax.fori_loop(0, _RPW, do_row, 0)


@jax.jit
def _topk_small_idx(x):
  mesh = plsc.VectorSubcoreMesh(core_axis_name="c", subcore_axis_name="s")
  return pl.kernel(
      _sc_body,
      out_type=jax.ShapeDtypeStruct((_ROWS, _K), jnp.int32),
      mesh=mesh,
      compiler_params=pltpu.CompilerParams(needs_layout_passes=False),
      scratch_types=[
          pltpu.VMEM((_N,), jnp.float32),
          pltpu.VMEM((_N + _L,), jnp.int32),
          pltpu.VMEM((_NBINS * _L,), jnp.int32),
          pltpu.VMEM((_K,), jnp.int32),
      ],
  )(x)


def kernel(x, k):
  del k
  return _topk_small_idx(x)


# speedup vs baseline: 3.4349x; 1.0554x over previous
import functools

import jax
import jax.numpy as jnp
from jax import lax
from jax.experimental import pallas as pl
from jax.experimental.pallas import tpu as pltpu
from jax.experimental.pallas import tpu_sc as plsc

_ROWS = 128
_N = 32768
_K = 256
_L = 16
_NV = _N // _L
_NWORK = 32
_RPW = _ROWS // _NWORK
_NBINS = 256


def _f32_key(v):
  bits = plsc.bitcast(v, jnp.int32)
  key = bits ^ ((bits >> 31) | jnp.int32(-2147483648))
  return plsc.bitcast(key, jnp.uint32)


def _lperm(x, perm):
  return jnp.take_along_axis(x, perm, axis=0)


def _sc_body(x_hbm, o_hbm, row_v, cand_v, hist_v, outi_v):
  cid = lax.axis_index("c")
  sid = lax.axis_index("s")
  wid = sid * 2 + cid
  lane = lax.iota(jnp.int32, _L)
  ones = jnp.ones((_L,), jnp.int32)
  zeros = jnp.zeros((_L,), jnp.int32)
  laneoff = lane * _NBINS

  def _popcnt(mask):
    return plsc.all_reduce_population_count(mask)[0]

  def _zero_hist(b, _):
    hist_v[pl.ds(b * _L, _L)] = zeros
    return 0

  lax.fori_loop(0, _NBINS, _zero_hist, 0)

  lane15 = jnp.full((_L,), 15, jnp.int32)

  def find_bin(need):
    need_v = zeros + need

    def p1(g, gtot):
      acc = zeros
      for t in range(_L):
        acc = acc + hist_v[pl.ds((g * _L + t) * _L, _L)]
      tsp = jnp.take_along_axis(plsc.cumsum(acc), lane15, axis=0)
      return jnp.where(lane == g, tsp, gtot)

    gtot = lax.fori_loop(0, _L, p1, zeros)
    gcs = plsc.cumsum(gtot)
    gxv = plsc.all_reduce_ffs(gcs >= need_v)
    gx = gxv[0]
    cum0_v = jnp.take_along_axis(gcs - gtot, gxv, axis=0)

    def p2(g, btot):
      def resolve():
        upd = btot
        for t in range(_L):
          v = hist_v[pl.ds((g * _L + t) * _L, _L)]
          tsp = jnp.take_along_axis(plsc.cumsum(v), lane15, axis=0)
          upd = jnp.where(lane == t, tsp, upd)
        return upd

      upd = lax.cond(g == gx, resolve, lambda: btot)
      for t in range(_L):
        hist_v[pl.ds((g * _L + t) * _L, _L)] = zeros
      return upd

    btot = lax.fori_loop(0, _L, p2, zeros)
    bcs = plsc.cumsum(btot)
    bfv = plsc.all_reduce_ffs((cum0_v + bcs) >= need_v)
    b0 = gx * _L + bfv[0]
    nbelow_v = cum0_v + jnp.take_along_axis(bcs - btot, bfv, axis=0)
    return b0, nbelow_v[0]

  def do_row(r, _):
    row = wid * _RPW + r
    with jax.named_scope("stage_dma_in"):
      pltpu.sync_copy(x_hbm.at[row], row_v)

    with jax.named_scope("stage_pa"):
      @plsc.parallel_loop(0, _NV, unroll=4)
      def pa(i):
        key = _f32_key(row_v[pl.ds(i * _L, _L)])
        dig = (key >> 24).astype(jnp.int32)
        plsc.addupdate_scatter(hist_v, [dig * _L + lane], ones)
    with jax.named_scope("stage_findbin"):
      b0, _ = find_bin(jnp.int32(_K))

    with jax.named_scope("stage_pb"):
      @plsc.parallel_loop(0, _NV, unroll=4, carry=jnp.int32(0))
      def pb(i, w):
        key = _f32_key(row_v[pl.ds(i * _L, _L)])
        keep = (key >> 24).astype(jnp.int32) <= b0
        idxv = i * _L + lane
        plsc.store_compressed(cand_v.at[pl.ds(w, _L)], idxv, mask=keep)
        return w + _popcnt(keep)
      mcount = pb

    path = b0.astype(jnp.uint32)
    n_lt_full = jnp.int32(0)
    for shift in (16, 8, 0):
      nvec = (mcount + _L - 1) // _L

      def lv1(i, nlt, shift=shift, path=path, mcount=mcount):
        valid = (i * _L + lane) < mcount
        idxv = cand_v[pl.ds(i * _L, _L)]
        key = _f32_key(plsc.load_gather(row_v, [idxv], mask=valid))
        pre = key >> (shift + 8)
        is_lt = valid & (pre < path)
        is_eq = valid & (pre == path)
        dig = ((key >> shift) & 0xFF).astype(jnp.int32)
        plsc.addupdate_scatter(hist_v, [dig * _L + lane], ones, mask=is_eq)
        return nlt + _popcnt(is_lt)

      with jax.named_scope("stage_lv1"):
        n_lt = plsc.parallel_loop(0, nvec, unroll=2, carry=jnp.int32(0))(lv1)
      bl, nbelow = find_bin(_K - n_lt)
      path = (path << 8) | bl.astype(jnp.uint32)
      n_lt_full = n_lt + nbelow

      def lv2(i, w, shift=shift, path=path, mcount=mcount):
        valid = (i * _L + lane) < mcount
        idxv = cand_v[pl.ds(i * _L, _L)]
        key = _f32_key(plsc.load_gather(row_v, [idxv], mask=valid))
        keep = valid & ((key >> shift) <= path)
        plsc.store_compressed(cand_v.at[pl.ds(w, _L)], idxv, mask=keep)
        return w + _popcnt(keep)

      with jax.named_scope("stage_lv2"):
        mcount = plsc.parallel_loop(0, nvec, unroll=2, carry=jnp.int32(0))(lv2)

    tkey = path
    need_ties = _K - n_lt_full
    nvec = (mcount + _L - 1) // _L

    def trim(i, carry):
      w, teq = carry
      valid = (i * _L + lane) < mcount
      idxv = cand_v[pl.ds(i * _L, _L)]
      key = _f32_key(plsc.load_gather(row_v, [idxv], mask=valid))
      is_lt = valid & (key < tkey)
      is_eq = valid & (key == tkey)
      erank = plsc.cumsum(is_eq.astype(jnp.int32))
      keep = is_lt | (is_eq & ((teq + erank) <= need_ties))
      plsc.store_compressed(cand_v.at[pl.ds(w, _L)], idxv, mask=keep)
      return (w + _popcnt(keep), teq + _popcnt(is_eq))

    with jax.named_scope("stage_trim"):
      plsc.parallel_loop(0, nvec, unroll=2,
                         carry=(jnp.int32(0), jnp.int32(0)))(trim)

    kv = []
    iv = []
    for i in range(16):
      idxv = cand_v[pl.ds(i * _L, _L)]
      kv.append(_f32_key(plsc.load_gather(row_v, [idxv])))
      iv.append(idxv)

    def ce_vreg(i, j, bsz):
      ka, ia, kb, ib = kv[i], iv[i], kv[j], iv[j]
      if bsz >= 16:
        dirm = ((lane * 16 + i) & bsz) == 0
      else:
        dirm = ((i & bsz) == 0)
        dirm = jnp.full((_L,), dirm, jnp.bool_)
      lt = (ka < kb) | ((ka == kb) & (ia < ib))
      keep_a = lt == dirm
      kv[i] = jnp.where(keep_a, ka, kb)
      iv[i] = jnp.where(keep_a, ia, ib)
      kv[j] = jnp.where(keep_a, kb, ka)
      iv[j] = jnp.where(keep_a, ib, ia)

    def ce_lane(i, dd, bsz):
      perm = lane ^ dd
      k0, i0 = kv[i], iv[i]
      kp = _lperm(k0, perm)
      ip = _lperm(i0, perm)
      lt = (k0 < kp) | ((k0 == kp) & (i0 < ip))
      dirm = ((lane * 16 + i) & bsz) == 0
      wantmin = ((lane & dd) == 0) == dirm
      keep = lt == wantmin
      kv[i] = jnp.where(keep, k0, kp)
      iv[i] = jnp.where(keep, i0, ip)

    _sort_scope = jax.named_scope("stage_sort"); _sort_scope.__enter__()
    bsz = 2
    while bsz <= 256:
      d = bsz // 2
      while d >= 1:
        if d < 16:
          for i in range(16):
            if (i & d) == 0:
              ce_vreg(i, i | d, bsz)
        else:
          dd = d // 16
          for i in range(16):
            ce_lane(i, dd, bsz)
        d //= 2
      bsz *= 2

    with jax.named_scope("stage_out"):
      for i in range(16):
        plsc.store_scatter(outi_v, [lane * 16 + i], iv[i])
      pltpu.sync_copy(outi_v, o_hbm.at[row])
    return 0

  lax.fori_loop(0, _RPW, do_row, 0)


@jax.jit
def _topk_small_idx(x):
  mesh = plsc.VectorSubcoreMesh(core_axis_name="c", subcore_axis_name="s")
  return pl.kernel(
      _sc_body,
      out_type=jax.ShapeDtypeStruct((_ROWS, _K), jnp.int32),
      mesh=mesh,
      compiler_params=pltpu.CompilerParams(needs_layout_passes=False),
      scratch_types=[
          pltpu.VMEM((_N,), jnp.float32),
          pltpu.VMEM((_N + _L,), jnp.int32),
          pltpu.VMEM((_NBINS * _L,), jnp.int32),
          pltpu.VMEM((_K,), jnp.int32),
      ],
  )(x)


def kernel(x, k):
  del k
  return _topk_small_idx(x)
